# Initial kernel scaffold; baseline (speedup 1.0000x reference)
import jax
import jax.numpy as jnp
from jax.experimental import pallas as pl


def kernel(h_T, h_0, graph_index, Wi1, bi1, Wi2, bi2, Wj1, bj1, Wj2, bj2):
    raise NotImplementedError("write your pallas kernel here")


# speedup vs baseline: 5.4502x; 5.4502x over previous
import jax
import jax.numpy as jnp
from jax.experimental import pallas as pl
from jax.experimental.pallas import tpu as pltpu

N_GRAPH = 256
BLOCK = 2000


def _body(seg_ref, hT_ref, h0_ref, Wi1a_ref, Wi1b_ref, bi1_ref, Wi2_ref,
          bi2_ref, Wj1_ref, bj1_ref, Wj2_ref, bj2_ref, out_ref):
    x_t = hT_ref[...]
    x_0 = h0_ref[...]

    hi = jnp.maximum(
        jax.lax.dot_general(x_t, Wi1a_ref[...], (((1,), (0,)), ((), ())),
                            preferred_element_type=jnp.float32)
        + jax.lax.dot_general(x_0, Wi1b_ref[...], (((1,), (0,)), ((), ())),
                              preferred_element_type=jnp.float32)
        + bi1_ref[...], 0.0)
    gate = jax.nn.sigmoid(
        jax.lax.dot_general(hi, Wi2_ref[...], (((1,), (0,)), ((), ())),
                            preferred_element_type=jnp.float32)
        + bi2_ref[...])

    hj = jnp.maximum(
        jax.lax.dot_general(x_t, Wj1_ref[...], (((1,), (0,)), ((), ())),
                            preferred_element_type=jnp.float32)
        + bj1_ref[...], 0.0)
    jv = (jax.lax.dot_general(hj, Wj2_ref[...], (((1,), (0,)), ((), ())),
                              preferred_element_type=jnp.float32)
          + bj2_ref[...])

    r_v = gate * jv

    seg = seg_ref[0, 0, :]
    onehot = (seg[:, None] == jax.lax.broadcasted_iota(
        jnp.int32, (BLOCK, N_GRAPH), 1)).astype(jnp.float32)
    partial = jax.lax.dot_general(onehot, r_v, (((0,), (0,)), ((), ())),
                                  preferred_element_type=jnp.float32)

    @pl.when(pl.program_id(0) == 0)
    def _init():
        out_ref[...] = jnp.zeros_like(out_ref)

    out_ref[...] += partial


def kernel(h_T, h_0, graph_index, Wi1, bi1, Wi2, bi2, Wj1, bj1, Wj2, bj2):
    n, d = h_T.shape
    grid = n // BLOCK
    seg3 = graph_index.reshape(grid, 1, BLOCK)
    Wi1a = Wi1[:d]
    Wi1b = Wi1[d:]

    row_spec = pl.BlockSpec((BLOCK, d), lambda i: (i, 0))
    full = lambda a: pl.BlockSpec(a.shape, lambda i: (0,) * a.ndim)

    return pl.pallas_call(
        _body,
        grid=(grid,),
        in_specs=[
            pl.BlockSpec((1, 1, BLOCK), lambda i: (i, 0, 0)),
            row_spec, row_spec,
            full(Wi1a), full(Wi1b), full(bi1.reshape(1, -1)), full(Wi2),
            full(bi2.reshape(1, -1)), full(Wj1), full(bj1.reshape(1, -1)),
            full(Wj2), full(bj2.reshape(1, -1)),
        ],
        out_specs=pl.BlockSpec((N_GRAPH, d), lambda i: (0, 0)),
        out_shape=jax.ShapeDtypeStruct((N_GRAPH, d), jnp.float32),
        compiler_params=pltpu.CompilerParams(
            dimension_semantics=("arbitrary",)),
    )(seg3, h_T, h_0, Wi1a, Wi1b, bi1.reshape(1, -1), Wi2,
      bi2.reshape(1, -1), Wj1, bj1.reshape(1, -1), Wj2, bj2.reshape(1, -1))
